# baseline (speedup 1.0000x reference)
import jax
import jax.numpy as jnp
from jax import lax
from jax.experimental import pallas as pl
from jax.experimental.pallas import tpu as pltpu
from jax.experimental.pallas import tpu_sc as plsc

NC = 2
NS = 16
NW = NC * NS
ROWS_PER_CHUNK = 4


def kernel(input_batch, table):
    batch, seq = input_batch.shape
    d_model = table.shape[1]
    idx = input_batch.astype(jnp.int32)

    rows_per_w = batch // NW
    assert rows_per_w * NW == batch

    mesh = plsc.VectorSubcoreMesh(core_axis_name="c", subcore_axis_name="s")

    @pl.kernel(
        mesh=mesh,
        out_type=jax.ShapeDtypeStruct((batch, seq, d_model), table.dtype),
        compiler_params=pltpu.CompilerParams(use_tc_tiling_on_sc=False),
        scratch_types=[
            pltpu.VMEM((seq,), jnp.int32),
            pltpu.VMEM((seq, d_model), table.dtype),
            pltpu.SemaphoreType.DMA,
        ],
    )
    def gather_kernel(table_hbm, idx_hbm, out_hbm, idx_v, rows_v, sem):
        wid = lax.axis_index("s") * NC + lax.axis_index("c")
        base = wid * rows_per_w

        @pl.loop(0, rows_per_w)
        def _(i):
            row = base + i
            pltpu.sync_copy(idx_hbm.at[row], idx_v)
            pltpu.async_copy(table_hbm.at[idx_v], rows_v, sem).wait()
            pltpu.sync_copy(rows_v, out_hbm.at[row])

    return gather_kernel(table, idx)


# speedup vs baseline: 1.1246x; 1.1246x over previous
import jax
import jax.numpy as jnp
from jax import lax
from jax.experimental import pallas as pl
from jax.experimental.pallas import tpu as pltpu
from jax.experimental.pallas import tpu_sc as plsc

NC = 2
NS = 16
NW = NC * NS


def kernel(input_batch, table):
    batch, seq = input_batch.shape
    d_model = table.shape[1]

    rows_per_w = batch // NW
    assert rows_per_w * NW == batch and rows_per_w % 2 == 0

    mesh = plsc.VectorSubcoreMesh(core_axis_name="c", subcore_axis_name="s")

    @pl.kernel(
        mesh=mesh,
        out_type=jax.ShapeDtypeStruct((batch, seq, d_model), table.dtype),
        compiler_params=pltpu.CompilerParams(use_tc_tiling_on_sc=False),
        scratch_types=[
            pltpu.VMEM((rows_per_w, seq), jnp.int32),
            pltpu.VMEM((seq, d_model), table.dtype),
            pltpu.VMEM((seq, d_model), table.dtype),
            pltpu.SemaphoreType.DMA,
            pltpu.SemaphoreType.DMA,
        ],
    )
    def gather_kernel(table_hbm, idx_hbm, out_hbm, idx_v, r0, r1, sg0, sg1):
        wid = lax.axis_index("s") * NC + lax.axis_index("c")
        base = wid * rows_per_w

        pltpu.sync_copy(idx_hbm.at[pl.ds(base, rows_per_w)], idx_v)

        pltpu.async_copy(table_hbm.at[idx_v.at[0]], r0, sg0)
        pltpu.async_copy(table_hbm.at[idx_v.at[1]], r1, sg1)

        @pl.loop(0, rows_per_w // 2 - 1)
        def _(j):
            i0 = 2 * j
            pltpu.make_async_copy(table_hbm.at[idx_v.at[i0]], r0, sg0).wait()
            pltpu.sync_copy(r0, out_hbm.at[base + i0])
            pltpu.async_copy(table_hbm.at[idx_v.at[i0 + 2]], r0, sg0)
            pltpu.make_async_copy(table_hbm.at[idx_v.at[i0 + 1]], r1, sg1).wait()
            pltpu.sync_copy(r1, out_hbm.at[base + i0 + 1])
            pltpu.async_copy(table_hbm.at[idx_v.at[i0 + 3]], r1, sg1)

        pltpu.make_async_copy(table_hbm.at[idx_v.at[rows_per_w - 2]], r0, sg0).wait()
        pltpu.sync_copy(r0, out_hbm.at[base + rows_per_w - 2])
        pltpu.make_async_copy(table_hbm.at[idx_v.at[rows_per_w - 1]], r1, sg1).wait()
        pltpu.sync_copy(r1, out_hbm.at[base + rows_per_w - 1])

    return gather_kernel(table, input_batch)
